# Initial kernel scaffold; baseline (speedup 1.0000x reference)
import functools

import jax
import jax.numpy as jnp
from jax import lax
from jax.experimental import pallas as pl
from jax.experimental.pallas import tpu as pltpu
from jax.experimental.pallas import tpu_sc as plsc

N = 10000
E = 320000
D = 128
G = 128

NC = 2
NS = 16
K = 100
C = 100
RPT = N // NS
ZR = 125


def _sc_segment_sum(h, src4, dst4):
    mesh = plsc.VectorSubcoreMesh(core_axis_name="c", subcore_axis_name="s")

    @functools.partial(
        pl.kernel,
        out_type=jax.ShapeDtypeStruct((NC, N, D), jnp.float32),
        mesh=mesh,
        scratch_types=[
            pltpu.VMEM((C, K), jnp.int32),
            pltpu.VMEM((C, K), jnp.int32),
            pltpu.VMEM((K, D), jnp.float32),
            pltpu.VMEM((K, D), jnp.float32),
            pltpu.VMEM((ZR, D), jnp.float32),
            pltpu.VMEM_SHARED((N, D), jnp.float32),
            pltpu.SemaphoreType.DMA,
            pltpu.SemaphoreType.DMA,
        ],
    )
    def seg_sum(h_hbm, src_hbm, dst_hbm, out_hbm,
                src_v, dst_v, buf0, buf1, zbuf, agg_sh, sem0, sem1):
        c = lax.axis_index("c")
        s = lax.axis_index("s")

        def zrow(i, carry):
            for j in range(D // 16):
                zbuf[i, pl.ds(j * 16, 16)] = jnp.zeros((16,), jnp.float32)
            return carry
        lax.fori_loop(0, ZR, zrow, 0)
        for t in range(RPT // ZR):
            pltpu.sync_copy(zbuf, agg_sh.at[pl.ds(s * RPT + t * ZR, ZR)])

        pltpu.sync_copy(src_hbm.at[c, s], src_v)
        pltpu.sync_copy(dst_hbm.at[c, s], dst_v)
        plsc.subcore_barrier()

        def body(cc, carry):
            c0 = 2 * cc
            c1 = 2 * cc + 1
            d0 = pltpu.async_copy(h_hbm.at[src_v.at[c0]], buf0, sem0)
            d1 = pltpu.async_copy(h_hbm.at[src_v.at[c1]], buf1, sem1)
            d0.wait()
            pltpu.sync_copy(buf0, agg_sh.at[dst_v.at[c0]], add=True)
            d1.wait()
            pltpu.sync_copy(buf1, agg_sh.at[dst_v.at[c1]], add=True)
            return carry
        lax.fori_loop(0, C // 2, body, 0)
        plsc.subcore_barrier()

        pltpu.sync_copy(agg_sh.at[pl.ds(s * RPT, RPT)],
                        out_hbm.at[c, pl.ds(s * RPT, RPT)])

    return seg_sum(h, src4, dst4)


def _tc_layer(h, agg, w1t, b1, g1, be1, w2t, b2, g2, be2, eps11):
    def body(h_ref, a_ref, w1_ref, b1_ref, g1_ref, be1_ref,
             w2_ref, b2_ref, g2_ref, be2_ref, e_ref, out_ref):
        xb = h_ref[...] * e_ref[0, 0] + (a_ref[0] + a_ref[1])
        z = jnp.dot(xb, w1_ref[...], preferred_element_type=jnp.float32)
        z = z + b1_ref[...]
        m = jnp.mean(z, axis=0, keepdims=True)
        v = jnp.mean(z * z, axis=0, keepdims=True) - m * m
        z = g1_ref[...] * (z - m) * lax.rsqrt(v + 1e-5) + be1_ref[...]
        z = jnp.maximum(z, 0.0)
        z = jnp.dot(z, w2_ref[...], preferred_element_type=jnp.float32)
        z = z + b2_ref[...]
        m = jnp.mean(z, axis=0, keepdims=True)
        v = jnp.mean(z * z, axis=0, keepdims=True) - m * m
        z = g2_ref[...] * (z - m) * lax.rsqrt(v + 1e-5) + be2_ref[...]
        out_ref[...] = jnp.maximum(z, 0.0)

    return pl.pallas_call(
        body,
        out_shape=jax.ShapeDtypeStruct((N, D), jnp.float32),
    )(h, agg, w1t, b1, g1, be1, w2t, b2, g2, be2, eps11)


def _pool_mlp(h, batch2, w1t, b1, g, be, w2t, b2):
    def body(h_ref, bt_ref, w1_ref, b1_ref, g_ref, be_ref,
             w2_ref, b2_ref, out_ref):
        gid = lax.broadcasted_iota(jnp.int32, (1, G), 1)
        oh = (bt_ref[...] == gid).astype(jnp.float32)
        pooled = lax.dot_general(
            oh, h_ref[...], (((0,), (0,)), ((), ())),
            preferred_element_type=jnp.float32)
        z = jnp.dot(pooled, w1_ref[...], preferred_element_type=jnp.float32)
        z = z + b1_ref[...]
        m = jnp.mean(z, axis=0, keepdims=True)
        v = jnp.mean(z * z, axis=0, keepdims=True) - m * m
        z = g_ref[...] * (z - m) * lax.rsqrt(v + 1e-5) + be_ref[...]
        z = jnp.maximum(z, 0.0)
        out = jnp.dot(z, w2_ref[...], preferred_element_type=jnp.float32)
        out_ref[...] = out + b2_ref[...]

    out_dim = w2t.shape[1]
    return pl.pallas_call(
        body,
        out_shape=jax.ShapeDtypeStruct((G, out_dim), jnp.float32),
    )(h, batch2, w1t, b1, g, be, w2t, b2)


def kernel(x, edge_index, batch, params, mlp_params):
    src4 = edge_index[0].reshape(NC, NS, C, K)
    dst4 = edge_index[1].reshape(NC, NS, C, K)
    h = x
    for p in params:
        agg = _sc_segment_sum(h, src4, dst4)
        h = _tc_layer(
            h, agg,
            p['W1'].T, p['b1'].reshape(1, -1),
            p['g1'].reshape(1, -1), p['be1'].reshape(1, -1),
            p['W2'].T, p['b2'].reshape(1, -1),
            p['g2'].reshape(1, -1), p['be2'].reshape(1, -1),
            (1.0 + p['eps']).reshape(1, 1),
        )
    return _pool_mlp(
        h, batch.reshape(N, 1),
        mlp_params['W1'].T, mlp_params['b1'].reshape(1, -1),
        mlp_params['g'].reshape(1, -1), mlp_params['be'].reshape(1, -1),
        mlp_params['W2'].T, mlp_params['b2'].reshape(1, -1),
    )


# speedup vs baseline: 6.9337x; 6.9337x over previous
import functools

import jax
import jax.numpy as jnp
from jax import lax
from jax.experimental import pallas as pl
from jax.experimental.pallas import tpu as pltpu
from jax.experimental.pallas import tpu_sc as plsc

N = 10000
E = 320000
D = 128
G = 128

NC = 2
NS = 16
K = 128
C = 80
CS = 16
EP = NC * NS * C * K
NP = 10240
RPT = NP // NS


def _sc_segment_sum(h, src4, dst4, zrows):
    mesh = plsc.VectorSubcoreMesh(core_axis_name="c", subcore_axis_name="s")

    @functools.partial(
        pl.kernel,
        out_type=jax.ShapeDtypeStruct((NC, NP, D), jnp.float32),
        mesh=mesh,
        scratch_types=[
            pltpu.VMEM((CS, K), jnp.int32),
            pltpu.VMEM((CS, K), jnp.int32),
            pltpu.VMEM((K, D), jnp.float32),
            pltpu.VMEM((K, D), jnp.float32),
            pltpu.VMEM_SHARED((NP, D), jnp.float32),
            pltpu.SemaphoreType.DMA,
            pltpu.SemaphoreType.DMA,
        ],
    )
    def seg_sum(h_hbm, src_hbm, dst_hbm, z_hbm, out_hbm,
                src_v, dst_v, buf0, buf1, agg_sh, sem0, sem1):
        c = lax.axis_index("c")
        s = lax.axis_index("s")

        pltpu.sync_copy(z_hbm, agg_sh.at[pl.ds(s * RPT, RPT)])
        plsc.subcore_barrier()

        for q in range(C // CS):
            pltpu.sync_copy(src_hbm.at[c, s, pl.ds(q * CS, CS)], src_v)
            pltpu.sync_copy(dst_hbm.at[c, s, pl.ds(q * CS, CS)], dst_v)

            def body(i, carry):
                c0 = 2 * i
                c1 = 2 * i + 1
                d0 = pltpu.async_copy(h_hbm.at[src_v.at[c0]], buf0, sem0)
                d1 = pltpu.async_copy(h_hbm.at[src_v.at[c1]], buf1, sem1)
                d0.wait()
                pltpu.sync_copy(buf0, agg_sh.at[dst_v.at[c0]], add=True)
                d1.wait()
                pltpu.sync_copy(buf1, agg_sh.at[dst_v.at[c1]], add=True)
                return carry
            lax.fori_loop(0, CS // 2, body, 0)
        plsc.subcore_barrier()

        pltpu.sync_copy(agg_sh.at[pl.ds(s * RPT, RPT)],
                        out_hbm.at[c, pl.ds(s * RPT, RPT)])

    return seg_sum(h, src4, dst4, zrows)


def _tc_layer(h, agg, w1t, b1, g1, be1, w2t, b2, g2, be2, eps11):
    def body(h_ref, a_ref, w1_ref, b1_ref, g1_ref, be1_ref,
             w2_ref, b2_ref, g2_ref, be2_ref, e_ref, out_ref):
        xb = h_ref[...] * e_ref[0, 0] + (a_ref[0, :N] + a_ref[1, :N])
        z = jnp.dot(xb, w1_ref[...], preferred_element_type=jnp.float32,
                    precision=lax.Precision.HIGHEST)
        z = z + b1_ref[...]
        m = jnp.mean(z, axis=0, keepdims=True)
        v = jnp.mean(z * z, axis=0, keepdims=True) - m * m
        z = g1_ref[...] * (z - m) * lax.rsqrt(v + 1e-5) + be1_ref[...]
        z = jnp.maximum(z, 0.0)
        z = jnp.dot(z, w2_ref[...], preferred_element_type=jnp.float32,
                    precision=lax.Precision.HIGHEST)
        z = z + b2_ref[...]
        m = jnp.mean(z, axis=0, keepdims=True)
        v = jnp.mean(z * z, axis=0, keepdims=True) - m * m
        z = g2_ref[...] * (z - m) * lax.rsqrt(v + 1e-5) + be2_ref[...]
        out_ref[...] = jnp.maximum(z, 0.0)

    return pl.pallas_call(
        body,
        out_shape=jax.ShapeDtypeStruct((N, D), jnp.float32),
    )(h, agg, w1t, b1, g1, be1, w2t, b2, g2, be2, eps11)


def _pool_mlp(h, batch2, w1t, b1, g, be, w2t, b2):
    def body(h_ref, bt_ref, w1_ref, b1_ref, g_ref, be_ref,
             w2_ref, b2_ref, out_ref):
        gid = lax.broadcasted_iota(jnp.int32, (1, G), 1)
        oh = (bt_ref[...] == gid).astype(jnp.float32)
        pooled = lax.dot_general(
            oh, h_ref[...], (((0,), (0,)), ((), ())),
            preferred_element_type=jnp.float32,
                    precision=lax.Precision.HIGHEST)
        z = jnp.dot(pooled, w1_ref[...], preferred_element_type=jnp.float32,
                    precision=lax.Precision.HIGHEST)
        z = z + b1_ref[...]
        m = jnp.mean(z, axis=0, keepdims=True)
        v = jnp.mean(z * z, axis=0, keepdims=True) - m * m
        z = g_ref[...] * (z - m) * lax.rsqrt(v + 1e-5) + be_ref[...]
        z = jnp.maximum(z, 0.0)
        out = jnp.dot(z, w2_ref[...], preferred_element_type=jnp.float32,
                    precision=lax.Precision.HIGHEST)
        out_ref[...] = out + b2_ref[...]

    out_dim = w2t.shape[1]
    return pl.pallas_call(
        body,
        out_shape=jax.ShapeDtypeStruct((G, out_dim), jnp.float32),
    )(h, batch2, w1t, b1, g, be, w2t, b2)


def kernel(x, edge_index, batch, params, mlp_params):
    npad = EP - E
    pad_src = (jnp.arange(npad, dtype=jnp.int32) * 61) % N
    pad_dst = N + (jnp.arange(npad, dtype=jnp.int32) % (NP - N))
    src4 = jnp.concatenate([edge_index[0], pad_src]).reshape(NC, NS, C, K)
    dst4 = jnp.concatenate([edge_index[1], pad_dst]).reshape(NC, NS, C, K)
    zrows = jnp.zeros((RPT, D), jnp.float32)
    h = x
    for p in params:
        agg = _sc_segment_sum(h, src4, dst4, zrows)
        h = _tc_layer(
            h, agg,
            p['W1'].T, p['b1'].reshape(1, -1),
            p['g1'].reshape(1, -1), p['be1'].reshape(1, -1),
            p['W2'].T, p['b2'].reshape(1, -1),
            p['g2'].reshape(1, -1), p['be2'].reshape(1, -1),
            (1.0 + p['eps']).reshape(1, 1),
        )
    return _pool_mlp(
        h, batch.reshape(N, 1),
        mlp_params['W1'].T, mlp_params['b1'].reshape(1, -1),
        mlp_params['g'].reshape(1, -1), mlp_params['be'].reshape(1, -1),
        mlp_params['W2'].T, mlp_params['b2'].reshape(1, -1),
    )
